# baseline (speedup 1.0000x reference)
import functools

import jax
import jax.numpy as jnp
from jax import lax
from jax.experimental import pallas as pl
from jax.experimental.pallas import tpu as pltpu
from jax.experimental.pallas import tpu_sc as plsc

_NUM_TABLE_ROWS = 100000
_DIM = 300
_DIM_PAD = 304

_INFO = plsc.get_sparse_core_info()
_NC = _INFO.num_cores
_NS = _INFO.num_subcores
_NW = _NC * _NS

_B = 4096 * 50
_CHUNK = 128
_NCHUNKS = _B // _CHUNK
_CHUNKS_PER_W = _NCHUNKS // _NW


def _sc_body(x_hbm, table_hbm, out0, out1, out2, idx_v, rows_v, sem):
    cid = lax.axis_index("c")
    sid = lax.axis_index("s")
    wid = sid * _NC + cid
    outs = (out0, out1, out2)
    for t in range(3):
        def body(j, _, t=t):
            c = wid * _CHUNKS_PER_W + j
            pltpu.sync_copy(x_hbm.at[pl.ds(t * _B + c * _CHUNK, _CHUNK)], idx_v)
            pltpu.async_copy(table_hbm.at[idx_v], rows_v, sem).wait()
            pltpu.sync_copy(rows_v, outs[t].at[pl.ds(c * _CHUNK, _CHUNK)])
            return 0
        lax.fori_loop(0, _CHUNKS_PER_W, body, 0)


_gather = functools.partial(
    pl.kernel,
    out_type=(
        jax.ShapeDtypeStruct((_B, _DIM_PAD), jnp.float32),
        jax.ShapeDtypeStruct((_B, _DIM_PAD), jnp.float32),
        jax.ShapeDtypeStruct((_B, _DIM_PAD), jnp.float32),
    ),
    mesh=plsc.VectorSubcoreMesh(core_axis_name="c", subcore_axis_name="s"),
    scratch_types=[
        pltpu.VMEM((_CHUNK,), jnp.int32),
        pltpu.VMEM((_CHUNK, _DIM_PAD), jnp.float32),
        pltpu.SemaphoreType.DMA,
    ],
    compiler_params=pltpu.CompilerParams(use_tc_tiling_on_sc=False),
)(_sc_body)


def kernel(x, embedding_weight):
    table_pad = jnp.pad(embedding_weight, ((0, 0), (0, _DIM_PAD - _DIM)))
    o0, o1, o2 = _gather(x.reshape(-1), table_pad)
    shape = (4096, 50, _DIM)

    def unpad(o):
        return o[:, :_DIM].reshape(shape)

    return (unpad(o0), unpad(o1), unpad(o2))


# speedup vs baseline: 1.0921x; 1.0921x over previous
import functools

import jax
import jax.numpy as jnp
from jax import lax
from jax.experimental import pallas as pl
from jax.experimental.pallas import tpu as pltpu
from jax.experimental.pallas import tpu_sc as plsc

_NUM_TABLE_ROWS = 100000
_DIM = 300
_DIM_PAD = 304

_INFO = plsc.get_sparse_core_info()
_NC = _INFO.num_cores
_NS = _INFO.num_subcores
_NW = _NC * _NS

_B = 4096 * 50
_ROWS_PER_W = _B // _NW
_CHUNK = 80
_NBUF = 4
_CPW = _ROWS_PER_W // _CHUNK


def _make_body(t):
    def _sc_body(x_hbm, table_hbm, out, idx_v, rows, gsems, wsems):
        cid = lax.axis_index("c")
        sid = lax.axis_index("s")
        wid = sid * _NC + cid
        base = t * _B + wid * _ROWS_PER_W
        pltpu.sync_copy(x_hbm.at[pl.ds(base, _ROWS_PER_W)], idx_v)

        def g_copy(j, b):
            return pltpu.make_async_copy(
                table_hbm.at[idx_v.at[pl.ds(j * _CHUNK, _CHUNK)]],
                rows[b], gsems[b])

        def w_copy(j, b):
            return pltpu.make_async_copy(
                rows[b],
                out.at[pl.ds(wid * _ROWS_PER_W + j * _CHUNK, _CHUNK)],
                wsems[b])

        for b in range(_NBUF):
            g_copy(b, b).start()

        @pl.loop(0, _CPW, step=_NBUF)
        def _(j):
            for b in range(_NBUF):
                jj = j + b
                g_copy(jj, b).wait()
                w_copy(jj, b).start()

                @pl.when(jj + _NBUF < _CPW)
                def _():
                    w_copy(jj, b).wait()
                    g_copy(jj + _NBUF, b).start()

        for b in range(_NBUF):
            w_copy(_CPW - _NBUF + b, b).wait()

    return _sc_body


def _make_gather(t):
    return functools.partial(
        pl.kernel,
        out_type=jax.ShapeDtypeStruct((_B, _DIM_PAD), jnp.float32),
        mesh=plsc.VectorSubcoreMesh(core_axis_name="c", subcore_axis_name="s"),
        scratch_types=[
            pltpu.VMEM((_ROWS_PER_W,), jnp.int32),
            tuple(pltpu.VMEM((_CHUNK, _DIM_PAD), jnp.float32)
                  for _ in range(_NBUF)),
            tuple(pltpu.SemaphoreType.DMA for _ in range(_NBUF)),
            tuple(pltpu.SemaphoreType.DMA for _ in range(_NBUF)),
        ],
        compiler_params=pltpu.CompilerParams(use_tc_tiling_on_sc=False),
    )(_make_body(t))


_gathers = [_make_gather(t) for t in range(3)]


def kernel(x, embedding_weight):
    table_pad = jnp.pad(embedding_weight, ((0, 0), (0, _DIM_PAD - _DIM))) + 0.0
    xf = x.reshape(-1)
    shape = (4096, 50, _DIM)

    def unpad(o):
        return o[:, :_DIM].reshape(shape) + 0.0

    return tuple(unpad(_gathers[t](xf, table_pad)) for t in range(3))
